# baseline (speedup 1.0000x reference)
import jax
import jax.numpy as jnp
from jax import lax
from jax.experimental import pallas as pl
from jax.experimental.pallas import tpu as pltpu
from jax.experimental.pallas import tpu_sc as plsc

_SEQ = 8192
_D = 768
_B = 4

_NC = 2
_NS = 16
_NW = _NC * _NS
_ROWS_W = _SEQ // _NW
_R = 8
_NJ = _ROWS_W // _R
_LANES = 16


def _sc_body(in_hbm, tab_hbm, out_hbm, tab_v, io_v, tab_sem, in_sem, out_sem):
    wid = lax.axis_index("s") * _NC + lax.axis_index("c")
    rbase = wid * _ROWS_W

    def tab_copy(j, jp):
        return pltpu.make_async_copy(
            tab_hbm.at[pl.ds(rbase + j * _R, _R)], tab_v.at[jp], tab_sem.at[jp])

    def in_copy(j, p):
        r0 = rbase + j * _R
        return pltpu.make_async_copy(
            in_hbm.at[:, pl.ds(r0, _R)], io_v.at[p], in_sem.at[p])

    def out_copy(j, p):
        r0 = rbase + j * _R
        return pltpu.make_async_copy(
            io_v.at[p], out_hbm.at[:, pl.ds(r0, _R)], out_sem.at[p])

    tab_copy(0, 0).start()
    in_copy(0, 0).start()
    in_copy(1, 1).start()

    def slab(j, _):
        p = j % 4
        jp = j % 2

        @pl.when(j + 2 < _NJ)
        def _():
            @pl.when(j >= 2)
            def _():
                out_copy(j - 2, (j + 2) % 4).wait()

            in_copy(j + 2, (j + 2) % 4).start()

        @pl.when(j + 1 < _NJ)
        def _():
            tab_copy(j + 1, (j + 1) % 2).start()

        in_copy(j, p).wait()
        tab_copy(j, jp).wait()

        @plsc.parallel_loop(0, _R, step=1)
        def _(r):
            for b in range(_B):
                for c in range(0, _D, _LANES):
                    sl = pl.ds(c, _LANES)
                    io_v[p, b, r, sl] = io_v[p, b, r, sl] + tab_v[jp, r, sl]

        out_copy(j, p).start()
        return 0

    lax.fori_loop(0, _NJ, slab, 0)

    out_copy(_NJ - 4, 0).wait()
    out_copy(_NJ - 3, 1).wait()
    out_copy(_NJ - 2, 2).wait()
    out_copy(_NJ - 1, 3).wait()


@jax.jit
def kernel(inputs, pos_table):
    mesh = plsc.VectorSubcoreMesh(core_axis_name="c", subcore_axis_name="s")
    k = pl.kernel(
        _sc_body,
        out_type=jax.ShapeDtypeStruct((_B, _SEQ, _D), jnp.float32),
        mesh=mesh,
        scratch_types=[
            pltpu.VMEM((2, _R, _D), jnp.float32),
            pltpu.VMEM((4, _B, _R, _D), jnp.float32),
            pltpu.SemaphoreType.DMA((2,)),
            pltpu.SemaphoreType.DMA((4,)),
            pltpu.SemaphoreType.DMA((4,)),
        ],
        compiler_params=pltpu.CompilerParams(
            disable_bounds_checks=True,
            disable_semaphore_checks=True,
        ),
    )
    return k(inputs, pos_table)


# speedup vs baseline: 1.7090x; 1.7090x over previous
import jax
import jax.numpy as jnp
from jax import lax
from jax.experimental import pallas as pl
from jax.experimental.pallas import tpu as pltpu
from jax.experimental.pallas import tpu_sc as plsc

_SEQ = 8192
_D = 768
_B = 4

_NC = 2
_NS = 16
_NW = _NC * _NS
_ROWS_W = _SEQ // _NW
_R = 16
_NJ = _ROWS_W // _R
_NITEMS = _NJ * _B
_LANES = 16


def _sc_body(in_hbm, tab_hbm, out_hbm, tab_v, in_v, out_v,
             tab_sem, in_sem, out_sem):
    wid = lax.axis_index("s") * _NC + lax.axis_index("c")
    rbase = wid * _ROWS_W

    def tab_copy(j, jp):
        return pltpu.make_async_copy(
            tab_hbm.at[pl.ds(rbase + j * _R, _R)], tab_v.at[jp], tab_sem.at[jp])

    def in_copy(t, p):
        r0 = rbase + (t // _B) * _R
        return pltpu.make_async_copy(
            in_hbm.at[t % _B, pl.ds(r0, _R)], in_v.at[p], in_sem.at[p])

    def out_copy(t, p):
        r0 = rbase + (t // _B) * _R
        return pltpu.make_async_copy(
            out_v.at[p], out_hbm.at[t % _B, pl.ds(r0, _R)], out_sem.at[p])

    tab_copy(0, 0).start()
    in_copy(0, 0).start()

    def item(t, _):
        j = t // _B
        b = t % _B
        p = t % 2
        jp = j % 2

        @pl.when(t + 1 < _NITEMS)
        def _():
            in_copy(t + 1, (t + 1) % 2).start()

        @pl.when((b == 0) & (j + 1 < _NJ))
        def _():
            tab_copy(j + 1, (j + 1) % 2).start()

        in_copy(t, p).wait()

        @pl.when(b == 0)
        def _():
            tab_copy(j, jp).wait()

        @pl.when(t >= 2)
        def _():
            out_copy(t - 2, p).wait()

        @plsc.parallel_loop(0, _R, step=1)
        def _(r):
            for c in range(0, _D, _LANES):
                sl = pl.ds(c, _LANES)
                out_v[p, r, sl] = in_v[p, r, sl] + tab_v[jp, r, sl]

        out_copy(t, p).start()
        return 0

    lax.fori_loop(0, _NITEMS, item, 0)

    out_copy(_NITEMS - 2, 0).wait()
    out_copy(_NITEMS - 1, 1).wait()


@jax.jit
def kernel(inputs, pos_table):
    mesh = plsc.VectorSubcoreMesh(core_axis_name="c", subcore_axis_name="s")
    k = pl.kernel(
        _sc_body,
        out_type=jax.ShapeDtypeStruct((_B, _SEQ, _D), jnp.float32),
        mesh=mesh,
        scratch_types=[
            pltpu.VMEM((2, _R, _D), jnp.float32),
            pltpu.VMEM((2, _R, _D), jnp.float32),
            pltpu.VMEM((2, _R, _D), jnp.float32),
            pltpu.SemaphoreType.DMA((2,)),
            pltpu.SemaphoreType.DMA((2,)),
            pltpu.SemaphoreType.DMA((2,)),
        ],
        compiler_params=pltpu.CompilerParams(
            disable_bounds_checks=True,
            disable_semaphore_checks=True,
        ),
    )
    return k(inputs, pos_table)
